# baseline (speedup 1.0000x reference)
import functools

import jax
import jax.numpy as jnp
from jax import lax
from jax.experimental import pallas as pl
from jax.experimental.pallas import tpu as pltpu
from jax.experimental.pallas import tpu_sc as plsc

_SEQ = 8192
_DIM = 1024
_BSZ = 4
_NC = 2
_NS = 16
_NW = _NC * _NS
_ROWS_PER_W = _SEQ // _NW
_CHUNK = 32
_NCHUNK = _ROWS_PER_W // _CHUNK
_NBUF = 3


@functools.partial(
    pl.kernel,
    out_type=jax.ShapeDtypeStruct((_BSZ, _SEQ, _DIM), jnp.float32),
    mesh=plsc.VectorSubcoreMesh(core_axis_name="c", subcore_axis_name="s"),
    scratch_types=(
        [pltpu.VMEM((_CHUNK, _DIM), jnp.float32)] * _NBUF
        + [pltpu.SemaphoreType.DMA] * (2 * _NBUF)
    ),
)
def _bcast_kernel(table_hbm, out_hbm, *scratch):
    bufs = scratch[:_NBUF]
    sins = scratch[_NBUF:2 * _NBUF]
    souts = scratch[2 * _NBUF:]
    wid = lax.axis_index("s") * _NC + lax.axis_index("c")
    base = wid * _ROWS_PER_W

    def start_load(c):
        row = base + c * _CHUNK
        return pltpu.async_copy(
            table_hbm.at[pl.ds(row, _CHUNK)], bufs[c % _NBUF], sins[c % _NBUF])

    def start_stores(c):
        row = base + c * _CHUNK
        return [
            pltpu.async_copy(
                bufs[c % _NBUF], out_hbm.at[b, pl.ds(row, _CHUNK)],
                souts[c % _NBUF])
            for b in range(_BSZ)
        ]

    loads = [None] * _NCHUNK
    stores = [None] * _NCHUNK
    for c in range(min(_NBUF, _NCHUNK)):
        loads[c] = start_load(c)
    for c in range(_NCHUNK):
        loads[c].wait()
        stores[c] = start_stores(c)
        nxt = c + _NBUF
        if nxt < _NCHUNK:
            for d in stores[nxt - _NBUF]:
                d.wait()
            loads[nxt] = start_load(nxt)
    for c in range(max(0, _NCHUNK - _NBUF), _NCHUNK):
        for d in stores[c]:
            d.wait()


def kernel(inputs, table):
    del inputs
    return _bcast_kernel(table)


# speedup vs baseline: 1.0142x; 1.0142x over previous
import functools

import jax
import jax.numpy as jnp
from jax import lax
from jax.experimental import pallas as pl
from jax.experimental.pallas import tpu as pltpu
from jax.experimental.pallas import tpu_sc as plsc

_SEQ = 8192
_DIM = 1024
_BSZ = 4
_NC = 2
_NS = 16
_NW = _NC * _NS
_ROWS_PER_W = _SEQ // _NW
_CHUNK = 40
_CHUNK_ROWS = [_CHUNK] * (_ROWS_PER_W // _CHUNK) + (
    [_ROWS_PER_W % _CHUNK] if _ROWS_PER_W % _CHUNK else [])
_CHUNK_BASE = [sum(_CHUNK_ROWS[:i]) for i in range(len(_CHUNK_ROWS))]
_NCHUNK = len(_CHUNK_ROWS)
_NBUF = 3


@functools.partial(
    pl.kernel,
    out_type=jax.ShapeDtypeStruct((_BSZ, _SEQ, _DIM), jnp.float32),
    mesh=plsc.VectorSubcoreMesh(core_axis_name="c", subcore_axis_name="s"),
    scratch_types=(
        [pltpu.VMEM((_CHUNK, _DIM), jnp.float32)] * _NBUF
        + [pltpu.SemaphoreType.DMA] * (2 * _NBUF)
    ),
)
def _bcast_kernel(table_hbm, out_hbm, *scratch):
    bufs = scratch[:_NBUF]
    sins = scratch[_NBUF:2 * _NBUF]
    souts = scratch[2 * _NBUF:]
    wid = lax.axis_index("s") * _NC + lax.axis_index("c")
    base = wid * _ROWS_PER_W

    def start_load(c):
        row = base + _CHUNK_BASE[c]
        n = _CHUNK_ROWS[c]
        return pltpu.async_copy(
            table_hbm.at[pl.ds(row, n)], bufs[c % _NBUF].at[pl.ds(0, n)],
            sins[c % _NBUF])

    def start_stores(c):
        row = base + _CHUNK_BASE[c]
        n = _CHUNK_ROWS[c]
        return [
            pltpu.async_copy(
                bufs[c % _NBUF].at[pl.ds(0, n)],
                out_hbm.at[b, pl.ds(row, n)],
                souts[c % _NBUF])
            for b in range(_BSZ)
        ]

    loads = [None] * _NCHUNK
    stores = [None] * _NCHUNK
    for c in range(min(_NBUF, _NCHUNK)):
        loads[c] = start_load(c)
    for c in range(_NCHUNK):
        loads[c].wait()
        stores[c] = start_stores(c)
        nxt = c + _NBUF
        if nxt < _NCHUNK:
            for d in stores[nxt - _NBUF]:
                d.wait()
            loads[nxt] = start_load(nxt)
    for c in range(max(0, _NCHUNK - _NBUF), _NCHUNK):
        for d in stores[c]:
            d.wait()


def kernel(inputs, table):
    del inputs
    return _bcast_kernel(table)
